# baseline (speedup 1.0000x reference)
import jax
import jax.numpy as jnp
from jax.experimental import pallas as pl
from jax.experimental.pallas import tpu as pltpu

_B = 16
_N = 128
_H = 256
_RB = 16


def _dot(a, b):
    return jnp.dot(a, b, preferred_element_type=jnp.float32)


def _decoder_kernel(z_ref, Wemb_ref, b_emb_h, Wg0_ref, bg0_h, Wg1_ref,
                    bg1_h, Wg2_ref, bg2_h, We1_ref, be1_h, w2r_h, b2_h,
                    out_ref, bias_v, b2_v, sem):
    f32 = jnp.float32
    copies = [
        pltpu.make_async_copy(b_emb_h, bias_v.at[0:1, :], sem.at[0]),
        pltpu.make_async_copy(bg0_h, bias_v.at[1:2, :], sem.at[1]),
        pltpu.make_async_copy(bg1_h, bias_v.at[2:3, :], sem.at[2]),
        pltpu.make_async_copy(bg2_h, bias_v.at[3:4, :], sem.at[3]),
        pltpu.make_async_copy(be1_h, bias_v.at[4:5, :], sem.at[4]),
        pltpu.make_async_copy(w2r_h, bias_v.at[5:6, :], sem.at[5]),
        pltpu.make_async_copy(b2_h, b2_v, sem.at[6]),
    ]
    for c in copies:
        c.start()

    z = z_ref[...]
    zw = _dot(z, Wemb_ref[...])

    ii = jax.lax.broadcasted_iota(jnp.int32, (_N, 1), 0).astype(f32)
    dis = jax.lax.rsqrt(ii + 1.0)
    r2 = jax.lax.broadcasted_iota(jnp.int32, (_N, _N), 0)
    c2 = jax.lax.broadcasted_iota(jnp.int32, (_N, _N), 1)
    disr = jax.lax.rsqrt(c2.astype(f32) + 1.0)
    S = jnp.where(c2 <= r2, dis * disr, 0.0)
    s = _dot(S, jnp.ones((_N, 1), f32))

    copies[0].wait()
    emb = zw + bias_v[0:1, :]

    copies[1].wait()
    bg0 = bias_v[1:2, :]
    E0 = _dot(emb, Wg0_ref[...]) + bg0
    y = jnp.maximum(E0, 0.0)
    x = jnp.maximum(s * (E0[0:1, :] - bg0) + bg0, 0.0)

    for li, Wr in ((2, Wg1_ref), (3, Wg2_ref)):
        copies[li].wait()
        b = bias_v[li:li + 1, :]
        m = _dot(jnp.concatenate([x, y], axis=0), Wr[...])
        x = jnp.maximum(_dot(S, m[0:_N, :]) + b, 0.0)
        y = jnp.maximum(m[_N:_N + _B, :] + b, 0.0)

    We1 = We1_ref[...]
    xy = jnp.concatenate([x, y], axis=0)
    At = _dot(xy, We1[0:_H, :])
    Bt = _dot(xy, We1[_H:2 * _H, :])
    A = At[0:_N, :]
    Bp = Bt[0:_N, :]

    copies[4].wait()
    copies[5].wait()
    copies[6].wait()
    be1 = bias_v[4:5, :]
    w2r = bias_v[5:6, :]
    b2 = b2_v[...]

    ty = jnp.maximum(At[_N:_N + _B, :] + Bt[_N:_N + _B, :] + be1, 0.0)
    pv = jax.nn.sigmoid(jnp.sum(ty * w2r, axis=1, keepdims=True) + b2)
    offdiag = (r2 != c2)
    out_ref[pl.ds(1, _B - 1), :, :] = jnp.where(
        offdiag[None, :, :], pv[1:_B].reshape(_B - 1, 1, 1), 0.0)

    Ab1 = A + be1
    w2b = w2r[None, :, :]
    rows = []
    for blk in range(_N // _RB):
        i0 = blk * _RB
        t = jnp.maximum(Ab1[i0:i0 + _RB, :][:, None, :] + Bp[None, :, :], 0.0)
        rows.append(jnp.sum(t * w2b, axis=-1))
    G = jnp.concatenate(rows, axis=0) + b2
    U = jnp.where(r2 < c2, jax.nn.sigmoid(G), 0.0)
    out_ref[0, :, :] = U + U.T


def kernel(z, W_emb, b_emb, W_gnn0, b_gnn0, W_gnn1, b_gnn1, W_gnn2, b_gnn2,
           W_e1, b_e1, W_e2, b_e2):
    f32 = jnp.float32
    args = (
        z, W_emb, b_emb.reshape(1, -1),
        W_gnn0, b_gnn0.reshape(1, -1),
        W_gnn1, b_gnn1.reshape(1, -1),
        W_gnn2, b_gnn2.reshape(1, -1),
        W_e1, b_e1.reshape(1, -1),
        W_e2.reshape(1, -1), b_e2.reshape(1, 1),
    )
    vmem = pl.BlockSpec(memory_space=pltpu.VMEM)
    hbm = pl.BlockSpec(memory_space=pltpu.HBM)
    in_specs = [vmem, vmem, hbm, vmem, hbm, vmem, hbm, vmem, hbm, vmem,
                hbm, hbm, hbm]
    return pl.pallas_call(
        _decoder_kernel,
        out_shape=jax.ShapeDtypeStruct((_B, _N, _N), f32),
        in_specs=in_specs,
        out_specs=pl.BlockSpec(memory_space=pltpu.VMEM),
        scratch_shapes=[
            pltpu.VMEM((6, _H), f32),
            pltpu.VMEM((1, 1), f32),
            pltpu.SemaphoreType.DMA((7,)),
        ],
    )(*args)


# speedup vs baseline: 1.1174x; 1.1174x over previous
import jax
import jax.numpy as jnp
from jax.experimental import pallas as pl

_B = 16
_N = 128
_H = 256
_RB = 32


def _dot(a, b):
    return jnp.dot(a, b, preferred_element_type=jnp.float32)


def _decoder_kernel(z_ref, Wemb_ref, b_emb_ref, Wg0_ref, bg0_ref, Wg1_ref,
                    bg1_ref, Wg2_ref, bg2_ref, We1_ref, be1_ref, w2r_ref,
                    b2_ref, out_ref):
    f32 = jnp.float32
    z = z_ref[...]
    emb = _dot(z, Wemb_ref[...]) + b_emb_ref[...]

    ii = jax.lax.broadcasted_iota(jnp.int32, (_N, 1), 0).astype(f32)
    dis = jax.lax.rsqrt(ii + 1.0)
    r2 = jax.lax.broadcasted_iota(jnp.int32, (_N, _N), 0)
    c2 = jax.lax.broadcasted_iota(jnp.int32, (_N, _N), 1)
    disr = jax.lax.rsqrt(c2.astype(f32) + 1.0)
    S = jnp.where(c2 <= r2, dis * disr, 0.0)
    s = _dot(S, jnp.ones((_N, 1), f32))

    bg0 = bg0_ref[...]
    E0 = _dot(emb, Wg0_ref[...]) + bg0
    y = jnp.maximum(E0, 0.0)
    x = jnp.maximum(s * (E0[0:1, :] - bg0) + bg0, 0.0)

    for Wr, br in ((Wg1_ref, bg1_ref), (Wg2_ref, bg2_ref)):
        b = br[...]
        m = _dot(jnp.concatenate([x, y], axis=0), Wr[...])
        x = jnp.maximum(_dot(S, m[0:_N, :]) + b, 0.0)
        y = jnp.maximum(m[_N:_N + _B, :] + b, 0.0)

    We1 = We1_ref[...]
    be1 = be1_ref[...]
    w2r = w2r_ref[...]
    b2 = b2_ref[...]
    xy = jnp.concatenate([x, y], axis=0)
    At = _dot(xy, We1[0:_H, :])
    Bt = _dot(xy, We1[_H:2 * _H, :])
    A = At[0:_N, :]
    Bp = Bt[0:_N, :]

    ty = jnp.maximum(At[_N:_N + _B, :] + Bt[_N:_N + _B, :] + be1, 0.0)
    pv = jax.nn.sigmoid(jnp.sum(ty * w2r, axis=1, keepdims=True) + b2)
    offdiag = (r2 != c2)
    out_ref[pl.ds(1, _B - 1), :, :] = jnp.where(
        offdiag[None, :, :], pv[1:_B].reshape(_B - 1, 1, 1), 0.0)

    Ab1 = A + be1
    w2b = w2r[None, :, :]
    rows = []
    for blk in range(_N // _RB):
        i0 = blk * _RB
        t = jnp.maximum(Ab1[i0:i0 + _RB, :][:, None, :] + Bp[None, :, :], 0.0)
        rows.append(jnp.sum(t * w2b, axis=-1))
    G = jnp.concatenate(rows, axis=0) + b2
    U = jnp.where(r2 < c2, jax.nn.sigmoid(G), 0.0)
    out_ref[0, :, :] = U + U.T


def kernel(z, W_emb, b_emb, W_gnn0, b_gnn0, W_gnn1, b_gnn1, W_gnn2, b_gnn2,
           W_e1, b_e1, W_e2, b_e2):
    args = (
        z, W_emb, b_emb.reshape(1, -1),
        W_gnn0, b_gnn0.reshape(1, -1),
        W_gnn1, b_gnn1.reshape(1, -1),
        W_gnn2, b_gnn2.reshape(1, -1),
        W_e1, b_e1.reshape(1, -1),
        W_e2.reshape(1, -1), b_e2.reshape(1, 1),
    )
    return pl.pallas_call(
        _decoder_kernel,
        out_shape=jax.ShapeDtypeStruct((_B, _N, _N), jnp.float32),
    )(*args)


# speedup vs baseline: 1.3004x; 1.1638x over previous
import jax
import jax.numpy as jnp
from jax.experimental import pallas as pl

_B = 16
_N = 128
_H = 256
_RB = 32


def _dot(a, b):
    return jnp.dot(a, b, preferred_element_type=jnp.float32)


def _decoder_kernel(z_ref, Wemb_ref, b_emb_ref, Wg0_ref, bg0_ref, Wg1_ref,
                    bg1_ref, Wg2_ref, bg2_ref, We1_ref, be1_ref, w2r_ref,
                    b2_ref, out_ref):
    f32 = jnp.float32
    z = z_ref[...]
    emb = _dot(z, Wemb_ref[...]) + b_emb_ref[...]

    ii = jax.lax.broadcasted_iota(jnp.int32, (_N, 1), 0).astype(f32)
    dis = jax.lax.rsqrt(ii + 1.0)
    r2 = jax.lax.broadcasted_iota(jnp.int32, (_N, _N), 0)
    c2 = jax.lax.broadcasted_iota(jnp.int32, (_N, _N), 1)
    disr = jax.lax.rsqrt(c2.astype(f32) + 1.0)
    S = jnp.where(c2 <= r2, dis * disr, 0.0)
    s = _dot(S, jnp.ones((_N, 1), f32))

    bg0 = bg0_ref[...]
    E0 = _dot(emb, Wg0_ref[...]) + bg0
    y = jnp.maximum(E0, 0.0)
    x = jnp.maximum(s * (E0[0:1, :] - bg0) + bg0, 0.0)

    for Wr, br in ((Wg1_ref, bg1_ref), (Wg2_ref, bg2_ref)):
        b = br[...]
        m = _dot(jnp.concatenate([x, y], axis=0), Wr[...])
        x = jnp.maximum(_dot(S, m[0:_N, :]) + b, 0.0)
        y = jnp.maximum(m[_N:_N + _B, :] + b, 0.0)

    We1 = We1_ref[...]
    be1 = be1_ref[...]
    w2r = w2r_ref[...]
    b2 = b2_ref[...]
    xy = jnp.concatenate([x, y], axis=0)
    At = _dot(xy, We1[0:_H, :])
    Bt = _dot(xy, We1[_H:2 * _H, :])
    A = At[0:_N, :]
    Bp = Bt[0:_N, :]

    ty = jnp.maximum(At[_N:_N + _B, :] + Bt[_N:_N + _B, :] + be1, 0.0)
    pv = jax.nn.sigmoid(jnp.sum(ty * w2r, axis=1, keepdims=True) + b2)
    offdiag = (r2 != c2)
    out_ref[pl.ds(1, _B - 1), :, :] = jnp.where(
        offdiag[None, :, :], pv[1:_B].reshape(_B - 1, 1, 1), 0.0)

    Ab1 = A + be1
    w2b = w2r[None, :, :]
    rows = []
    for blk in range(_N // _RB):
        i0 = blk * _RB
        t = jnp.maximum(
            Ab1[i0:i0 + _RB, :][:, None, :] + Bp[i0:, :][None, :, :], 0.0)
        part = jnp.sum(t * w2b, axis=-1)
        if i0:
            part = jnp.concatenate(
                [jnp.zeros((_RB, i0), jnp.float32), part], axis=1)
        rows.append(part)
    G = jnp.concatenate(rows, axis=0) + b2
    U = jnp.where(r2 < c2, jax.nn.sigmoid(G), 0.0)
    out_ref[0, :, :] = U + U.T


def kernel(z, W_emb, b_emb, W_gnn0, b_gnn0, W_gnn1, b_gnn1, W_gnn2, b_gnn2,
           W_e1, b_e1, W_e2, b_e2):
    args = (
        z, W_emb, b_emb.reshape(1, -1),
        W_gnn0, b_gnn0.reshape(1, -1),
        W_gnn1, b_gnn1.reshape(1, -1),
        W_gnn2, b_gnn2.reshape(1, -1),
        W_e1, b_e1.reshape(1, -1),
        W_e2.reshape(1, -1), b_e2.reshape(1, 1),
    )
    return pl.pallas_call(
        _decoder_kernel,
        out_shape=jax.ShapeDtypeStruct((_B, _N, _N), jnp.float32),
    )(*args)


# speedup vs baseline: 1.3068x; 1.0049x over previous
import jax
import jax.numpy as jnp
from jax.experimental import pallas as pl

_B = 16
_N = 128
_H = 256
_RB = 32


def _dot(a, b):
    return jnp.dot(a, b, preferred_element_type=jnp.float32)


def _decoder_kernel(z_ref, Wemb_ref, b_emb_ref, Wg0_ref, bg0_ref, Wg1_ref,
                    bg1_ref, Wg2_ref, bg2_ref, We1_ref, be1_ref, w2r_ref,
                    b2_ref, out_ref):
    f32 = jnp.float32
    z = z_ref[...]
    emb = _dot(z, Wemb_ref[...]) + b_emb_ref[...]

    ii = jax.lax.broadcasted_iota(jnp.int32, (_N, 1), 0).astype(f32)
    dis = jax.lax.rsqrt(ii + 1.0)
    r2 = jax.lax.broadcasted_iota(jnp.int32, (_N, _N), 0)
    c2 = jax.lax.broadcasted_iota(jnp.int32, (_N, _N), 1)
    disr = jax.lax.rsqrt(c2.astype(f32) + 1.0)
    S = jnp.where(c2 <= r2, dis * disr, 0.0)
    s = _dot(S, jnp.ones((_N, 1), f32))

    bg0 = bg0_ref[...]
    E0 = _dot(emb, Wg0_ref[...]) + bg0
    y = jnp.maximum(E0, 0.0)
    x = jnp.maximum(s * (E0[0:1, :] - bg0) + bg0, 0.0)

    for Wr, br in ((Wg1_ref, bg1_ref), (Wg2_ref, bg2_ref)):
        b = br[...]
        m = _dot(jnp.concatenate([x, y], axis=0), Wr[...])
        x = jnp.maximum(_dot(S, m[0:_N, :]) + b, 0.0)
        y = jnp.maximum(m[_N:_N + _B, :] + b, 0.0)

    We1 = We1_ref[...]
    be1 = be1_ref[...]
    w2r = w2r_ref[...]
    b2 = b2_ref[...]
    xy = jnp.concatenate([x, y], axis=0)
    At = _dot(xy, We1[0:_H, :])
    Bt = _dot(xy, We1[_H:2 * _H, :])
    A = At[0:_N, :]
    Bp = Bt[0:_N, :]

    ty = jnp.maximum(At[_N:_N + _B, :] + Bt[_N:_N + _B, :] + be1, 0.0)
    pv = jax.nn.sigmoid(jnp.sum(ty * w2r, axis=1, keepdims=True) + b2)
    offdiag = (r2 != c2)
    out_ref[pl.ds(1, _B - 1), :, :] = jnp.where(
        offdiag[None, :, :], pv[1:_B].reshape(_B - 1, 1, 1), 0.0)

    Ab1 = A + be1
    w2c = w2r.reshape(_H, 1)
    rows = []
    for blk in range(_N // _RB):
        i0 = blk * _RB
        t = jnp.maximum(
            Ab1[i0:i0 + _RB, :][:, None, :] + Bp[i0:, :][None, :, :], 0.0)
        part = _dot(t.reshape(-1, _H), w2c).reshape(_RB, _N - i0)
        if i0:
            part = jnp.concatenate(
                [jnp.zeros((_RB, i0), jnp.float32), part], axis=1)
        rows.append(part)
    G = jnp.concatenate(rows, axis=0) + b2
    U = jnp.where(r2 < c2, jax.nn.sigmoid(G), 0.0)
    out_ref[0, :, :] = U + U.T


def kernel(z, W_emb, b_emb, W_gnn0, b_gnn0, W_gnn1, b_gnn1, W_gnn2, b_gnn2,
           W_e1, b_e1, W_e2, b_e2):
    args = (
        z, W_emb, b_emb.reshape(1, -1),
        W_gnn0, b_gnn0.reshape(1, -1),
        W_gnn1, b_gnn1.reshape(1, -1),
        W_gnn2, b_gnn2.reshape(1, -1),
        W_e1, b_e1.reshape(1, -1),
        W_e2.reshape(1, -1), b_e2.reshape(1, 1),
    )
    return pl.pallas_call(
        _decoder_kernel,
        out_shape=jax.ShapeDtypeStruct((_B, _N, _N), jnp.float32),
    )(*args)
